# baseline (speedup 1.0000x reference)
import functools

import jax
import jax.numpy as jnp
from jax import lax
from jax.experimental import pallas as pl
from jax.experimental.pallas import tpu as pltpu
from jax.experimental.pallas import tpu_sc as plsc

_K = 32
_N0 = 4096
_N1 = 200
_NW = 32
_LANES = _N0 // _NW
_PAIRS = _N1 // 2


def _prologue_body(mask_ref, logits_ref, idx_ref, tab_ref, idxt_ref):
    masked = jnp.where(mask_ref[...] == 0.0, jnp.float32(-50.0), logits_ref[...])
    m = jnp.max(masked, axis=1, keepdims=True)
    s = masked - m
    lp = s - jnp.log(jnp.sum(jnp.exp(s), axis=1, keepdims=True))
    tab_ref[...] = lp.T
    idxt_ref[...] = jnp.transpose(
        idx_ref[...].reshape(_NW, _LANES, _N1), (0, 2, 1)
    )


def _prologue(mask, logits, idx):
    return pl.pallas_call(
        _prologue_body,
        out_shape=(
            jax.ShapeDtypeStruct((_K, _K), jnp.float32),
            jax.ShapeDtypeStruct((_NW, _N1, _LANES), jnp.int32),
        ),
    )(mask, logits, idx)


def _sc_expand(tab_flat, idxt):
    mesh = plsc.VectorSubcoreMesh(core_axis_name="c", subcore_axis_name="s")

    @functools.partial(
        pl.kernel,
        mesh=mesh,
        out_type=jax.ShapeDtypeStruct((_N1, _K, _N0), jnp.float32),
        scratch_types=[
            pltpu.VMEM((_K * _K * 16,), jnp.float32),
            pltpu.VMEM((_N1 * _LANES,), jnp.int32),
            pltpu.VMEM((2, _K, _LANES), jnp.float32),
            pltpu.SemaphoreType.DMA,
            pltpu.SemaphoreType.DMA,
        ],
        compiler_params=pltpu.CompilerParams(
            use_tc_tiling_on_sc=True, needs_layout_passes=False
        ),
    )
    def k(tab_hbm, idxt_hbm, out_hbm, tab_v, idx_v, buf_v, o0, o1):
        osem = (o0, o1)
        wid = lax.axis_index("s") * 2 + lax.axis_index("c")
        lane0 = wid * _LANES
        pltpu.sync_copy(idxt_hbm.at[wid], idx_v)
        pltpu.sync_copy(tab_hbm, tab_v)
        iota = lax.iota(jnp.int32, 16)

        def out_slice(j):
            return out_hbm.at[j, :, pl.ds(lane0, _LANES)]

        def build(j, b):
            for lg in range(_LANES // 16):
                idxv = idx_v[pl.ds(j * _LANES + lg * 16, 16)]
                a0 = idxv * 16 + iota
                vals = [
                    plsc.load_gather(tab_v, [a0 + c * (_K * 16)])
                    for c in range(_K)
                ]
                for c in range(_K):
                    buf_v[b, c, pl.ds(lg * 16, 16)] = vals[c]

        def start_out(j, b):
            pltpu.async_copy(buf_v.at[b], out_slice(j), osem[b])

        def wait_out(j, b):
            pltpu.make_async_copy(buf_v.at[b], out_slice(j), osem[b]).wait()

        def pair(p, carry):
            for b in (0, 1):
                j = 2 * p + b

                @pl.when(p > 0)
                def _():
                    wait_out(j, b)

                build(j, b)
                start_out(j, b)
            return carry

        lax.fori_loop(0, _PAIRS, pair, 0)
        last = 2 * _PAIRS - 2
        wait_out(last, 0)
        wait_out(last + 1, 1)

    return k(tab_flat, idxt)


def kernel(prev_labels, mask, logits):
    tab, idxt = _prologue(
        mask.astype(jnp.float32), logits.astype(jnp.float32),
        prev_labels.astype(jnp.int32),
    )
    tab16 = jnp.broadcast_to(
        tab.reshape(_K * _K, 1), (_K * _K, 16)
    ).reshape(_K * _K * 16)
    out_t = _sc_expand(tab16, idxt.reshape(_NW, _N1 * _LANES))
    return out_t.transpose(2, 0, 1)


# speedup vs baseline: 1.0348x; 1.0348x over previous
import functools

import jax
import jax.numpy as jnp
from jax import lax
from jax.experimental import pallas as pl
from jax.experimental.pallas import tpu as pltpu
from jax.experimental.pallas import tpu_sc as plsc

_K = 32
_N0 = 4096
_N1 = 200
_NW = 32
_LANES = _N0 // _NW
_PAIRS = _N1 // 2


def _prologue_body(mask_ref, logits_ref, idx_ref, tab_ref, idxt_ref):
    masked = jnp.where(mask_ref[...] == 0.0, jnp.float32(-50.0), logits_ref[...])
    m = jnp.max(masked, axis=1, keepdims=True)
    s = masked - m
    lp = s - jnp.log(jnp.sum(jnp.exp(s), axis=1, keepdims=True))
    tab_ref[...] = lp.T
    idxt_ref[...] = jnp.transpose(
        idx_ref[...].reshape(_NW, _LANES, _N1), (0, 2, 1)
    )


def _prologue(mask, logits, idx):
    return pl.pallas_call(
        _prologue_body,
        out_shape=(
            jax.ShapeDtypeStruct((_K, _K), jnp.float32),
            jax.ShapeDtypeStruct((_NW, _N1, _LANES), jnp.int32),
        ),
    )(mask, logits, idx)


def _sc_expand(tab_flat, idxt):
    mesh = plsc.VectorSubcoreMesh(core_axis_name="c", subcore_axis_name="s")

    @functools.partial(
        pl.kernel,
        mesh=mesh,
        out_type=jax.ShapeDtypeStruct((_N1, _K, _N0), jnp.float32),
        scratch_types=[
            pltpu.VMEM((_K * _K,), jnp.float32),
            pltpu.VMEM((_N1 * _LANES,), jnp.int32),
            pltpu.VMEM((2, _K, _LANES), jnp.float32),
            pltpu.SemaphoreType.DMA,
            pltpu.SemaphoreType.DMA,
        ],
        compiler_params=pltpu.CompilerParams(
            use_tc_tiling_on_sc=True, needs_layout_passes=False
        ),
    )
    def k(tab_hbm, idxt_hbm, out_hbm, tab_v, idx_v, buf_v, o0, o1):
        osem = (o0, o1)
        wid = lax.axis_index("s") * 2 + lax.axis_index("c")
        lane0 = wid * _LANES
        pltpu.sync_copy(idxt_hbm.at[wid], idx_v)
        pltpu.sync_copy(tab_hbm, tab_v)

        def out_slice(j):
            return out_hbm.at[j, :, pl.ds(lane0, _LANES)]

        def build(j, b):
            for lg in range(_LANES // 16):
                idxv = idx_v[pl.ds(j * _LANES + lg * 16, 16)]
                vals = [
                    plsc.load_gather(tab_v, [idxv + c * _K]) for c in range(_K)
                ]
                for c in range(_K):
                    buf_v[b, c, pl.ds(lg * 16, 16)] = vals[c]

        def start_out(j, b):
            pltpu.async_copy(buf_v.at[b], out_slice(j), osem[b])

        def wait_out(j, b):
            pltpu.make_async_copy(buf_v.at[b], out_slice(j), osem[b]).wait()

        def pair(p, carry):
            for b in (0, 1):
                j = 2 * p + b

                @pl.when(p > 0)
                def _():
                    wait_out(j, b)

                build(j, b)
                start_out(j, b)
            return carry

        lax.fori_loop(0, _PAIRS, pair, 0)
        last = 2 * _PAIRS - 2
        wait_out(last, 0)
        wait_out(last + 1, 1)

    return k(tab_flat, idxt)


def kernel(prev_labels, mask, logits):
    tab, idxt = _prologue(
        mask.astype(jnp.float32), logits.astype(jnp.float32),
        prev_labels.astype(jnp.int32),
    )
    out_t = _sc_expand(tab.reshape(_K * _K), idxt.reshape(_NW, _N1 * _LANES))
    return out_t.transpose(2, 0, 1)
